# baseline (speedup 1.0000x reference)
import functools

import jax
import jax.numpy as jnp
from jax import lax
from jax.experimental import pallas as pl
from jax.experimental.pallas import tpu as pltpu
from jax.experimental.pallas import tpu_sc as plsc

EMBED_DIM = 32

_info = plsc.get_sparse_core_info()
_NC, _NS = _info.num_cores, _info.num_subcores
_NW = _NC * _NS

_CHUNK = 800


def _make_gather(B: int, D: int):
  assert B % _NW == 0
  b_per_w = B // _NW
  assert b_per_w % _CHUNK == 0
  n_chunks = b_per_w // _CHUNK
  assert n_chunks >= 4 and n_chunks % 2 == 0
  n_pairs = (n_chunks - 2) // 2
  mesh = plsc.VectorSubcoreMesh(core_axis_name="c", subcore_axis_name="s")

  @functools.partial(
      pl.kernel,
      mesh=mesh,
      out_type=jax.ShapeDtypeStruct((B, D), jnp.float32),
      compiler_params=pltpu.CompilerParams(use_tc_tiling_on_sc=False),
      scratch_types=[
          pltpu.VMEM((_CHUNK,), jnp.int32),
          pltpu.VMEM((_CHUNK,), jnp.int32),
          pltpu.VMEM((_CHUNK, D), jnp.float32),
          pltpu.VMEM((_CHUNK, D), jnp.float32),
          pltpu.SemaphoreType.DMA,
          pltpu.SemaphoreType.DMA,
          pltpu.SemaphoreType.DMA,
          pltpu.SemaphoreType.DMA,
      ],
  )
  def k(table_hbm, idx_hbm, out_hbm, idx0, idx1, rows0, rows1,
        gsem0, gsem1, osem0, osem1):
    wid = lax.axis_index("s") * _NC + lax.axis_index("c")
    base = wid * b_per_w

    def load_idx(j, idx_v):
      pltpu.sync_copy(idx_hbm.at[pl.ds(base + j * _CHUNK, _CHUNK)], idx_v)

    def start_gather(idx_v, rows_v, sem):
      pltpu.async_copy(table_hbm.at[idx_v], rows_v, sem)

    def wait_gather(idx_v, rows_v, sem):
      pltpu.make_async_copy(table_hbm.at[idx_v], rows_v, sem).wait()

    def start_out(j, rows_v, sem):
      pltpu.async_copy(rows_v, out_hbm.at[pl.ds(base + j * _CHUNK, _CHUNK)],
                       sem)

    def wait_out(j, rows_v, sem):
      pltpu.make_async_copy(rows_v,
                            out_hbm.at[pl.ds(base + j * _CHUNK, _CHUNK)],
                            sem).wait()

    load_idx(0, idx0)
    start_gather(idx0, rows0, gsem0)
    load_idx(1, idx1)
    start_gather(idx1, rows1, gsem1)
    wait_gather(idx0, rows0, gsem0)
    start_out(0, rows0, osem0)

    def body(g, carry):
      ja = 2 * g + 1
      wait_out(ja - 1, rows0, osem0)
      load_idx(ja + 1, idx0)
      start_gather(idx0, rows0, gsem0)
      wait_gather(idx1, rows1, gsem1)
      start_out(ja, rows1, osem1)
      wait_out(ja, rows1, osem1)
      load_idx(ja + 2, idx1)
      start_gather(idx1, rows1, gsem1)
      wait_gather(idx0, rows0, gsem0)
      start_out(ja + 1, rows0, osem0)
      return carry

    lax.fori_loop(0, n_pairs, body, 0)

    wait_out(n_chunks - 2, rows0, osem0)
    wait_gather(idx1, rows1, gsem1)
    start_out(n_chunks - 1, rows1, osem1)
    wait_out(n_chunks - 1, rows1, osem1)

  return k


def kernel(input_ids, table):
  batch, hist = input_ids.shape
  flat_ids = input_ids.reshape(batch * hist)
  out = _make_gather(batch * hist, EMBED_DIM)(table, flat_ids)
  return out.reshape(batch, hist, EMBED_DIM)


# speedup vs baseline: 1.0054x; 1.0054x over previous
import functools

import jax
import jax.numpy as jnp
from jax import lax
from jax.experimental import pallas as pl
from jax.experimental.pallas import tpu as pltpu
from jax.experimental.pallas import tpu_sc as plsc

EMBED_DIM = 32

_info = plsc.get_sparse_core_info()
_NC, _NS = _info.num_cores, _info.num_subcores
_NW = _NC * _NS

_CHUNK = 1600


def _make_gather(B: int, D: int):
  assert B % _NW == 0
  b_per_w = B // _NW
  assert b_per_w % _CHUNK == 0
  n_chunks = b_per_w // _CHUNK
  assert n_chunks >= 4 and n_chunks % 2 == 0
  n_pairs = (n_chunks - 2) // 2
  mesh = plsc.VectorSubcoreMesh(core_axis_name="c", subcore_axis_name="s")

  @functools.partial(
      pl.kernel,
      mesh=mesh,
      out_type=jax.ShapeDtypeStruct((B, D), jnp.float32),
      compiler_params=pltpu.CompilerParams(use_tc_tiling_on_sc=False),
      scratch_types=[
          pltpu.VMEM((_CHUNK,), jnp.int32),
          pltpu.VMEM((_CHUNK,), jnp.int32),
          pltpu.VMEM((_CHUNK, D), jnp.float32),
          pltpu.VMEM((_CHUNK, D), jnp.float32),
          pltpu.SemaphoreType.DMA,
          pltpu.SemaphoreType.DMA,
          pltpu.SemaphoreType.DMA,
          pltpu.SemaphoreType.DMA,
      ],
  )
  def k(table_hbm, idx_hbm, out_hbm, idx0, idx1, rows0, rows1,
        gsem0, gsem1, osem0, osem1):
    wid = lax.axis_index("s") * _NC + lax.axis_index("c")
    base = wid * b_per_w

    def load_idx(j, idx_v):
      pltpu.sync_copy(idx_hbm.at[pl.ds(base + j * _CHUNK, _CHUNK)], idx_v)

    def start_gather(idx_v, rows_v, sem):
      pltpu.async_copy(table_hbm.at[idx_v], rows_v, sem)

    def wait_gather(idx_v, rows_v, sem):
      pltpu.make_async_copy(table_hbm.at[idx_v], rows_v, sem).wait()

    def start_out(j, rows_v, sem):
      pltpu.async_copy(rows_v, out_hbm.at[pl.ds(base + j * _CHUNK, _CHUNK)],
                       sem)

    def wait_out(j, rows_v, sem):
      pltpu.make_async_copy(rows_v,
                            out_hbm.at[pl.ds(base + j * _CHUNK, _CHUNK)],
                            sem).wait()

    load_idx(0, idx0)
    start_gather(idx0, rows0, gsem0)
    load_idx(1, idx1)
    start_gather(idx1, rows1, gsem1)
    wait_gather(idx0, rows0, gsem0)
    start_out(0, rows0, osem0)

    def body(g, carry):
      ja = 2 * g + 1
      wait_out(ja - 1, rows0, osem0)
      load_idx(ja + 1, idx0)
      start_gather(idx0, rows0, gsem0)
      wait_gather(idx1, rows1, gsem1)
      start_out(ja, rows1, osem1)
      wait_out(ja, rows1, osem1)
      load_idx(ja + 2, idx1)
      start_gather(idx1, rows1, gsem1)
      wait_gather(idx0, rows0, gsem0)
      start_out(ja + 1, rows0, osem0)
      return carry

    lax.fori_loop(0, n_pairs, body, 0)

    wait_out(n_chunks - 2, rows0, osem0)
    wait_gather(idx1, rows1, gsem1)
    start_out(n_chunks - 1, rows1, osem1)
    wait_out(n_chunks - 1, rows1, osem1)

  return k


def kernel(input_ids, table):
  batch, hist = input_ids.shape
  flat_ids = input_ids.reshape(batch * hist)
  out = _make_gather(batch * hist, EMBED_DIM)(table, flat_ids)
  return out.reshape(batch, hist, EMBED_DIM)
